# baseline (speedup 1.0000x reference)
import functools

import jax
import jax.numpy as jnp
from jax import lax
from jax.experimental import pallas as pl
from jax.experimental.pallas import tpu as pltpu
from jax.experimental.pallas import tpu_sc as plsc

_INV06 = 1.0 / 0.6


def _ssilu(x):
    return x * jax.nn.sigmoid(x) * _INV06


def _tables_body(h_ref, wg_ref, wp_ref, g_ref, p_ref):
    hb = h_ref[...]
    g_ref[...] = jnp.dot(hb, wg_ref[...], preferred_element_type=jnp.float32)
    p_ref[...] = jnp.dot(hb, wp_ref[...], preferred_element_type=jnp.float32)


def _stats_body(gp_ref, ea_ref, wc_ref, b1_ref, out_ref):
    pre = gp_ref[...] + jnp.dot(ea_ref[...], wc_ref[...],
                                preferred_element_type=jnp.float32) + b1_ref[...]
    x = _ssilu(pre)
    s1 = jnp.sum(x, axis=0, keepdims=True)
    s2 = jnp.sum(x * x, axis=0, keepdims=True)

    @pl.when(pl.program_id(0) == 0)
    def _():
        out_ref[...] = jnp.zeros_like(out_ref)

    out_ref[...] += jnp.concatenate([s1, s2], axis=0)


def _edge_body(gp_ref, ea_ref, wc_ref, b1_ref, st_ref, gam_ref, bet_ref,
               w2_ref, b2_ref, out_ref, *, n_edges):
    mean = st_ref[0:1, :] * (1.0 / n_edges)
    var = st_ref[1:2, :] * (1.0 / n_edges) - mean * mean
    s = gam_ref[...] * jax.lax.rsqrt(var + 1e-5)
    t = bet_ref[...] - mean * s
    pre = gp_ref[...] + jnp.dot(ea_ref[...], wc_ref[...],
                                preferred_element_type=jnp.float32) + b1_ref[...]
    x = _ssilu(pre)
    xn = x * s + t
    out_ref[...] = _ssilu(jnp.dot(xn, w2_ref[...],
                                  preferred_element_type=jnp.float32) + b2_ref[...])


def _node_body(h_ref, p0_ref, p1_ref, wa_ref, wb_ref, b1_ref,
               w2_ref, b2_ref, out_ref):
    hb = h_ref[...]
    agg = (p0_ref[...] + p1_ref[...]) * 0.01
    y = _ssilu(jnp.dot(hb, wa_ref[...], preferred_element_type=jnp.float32)
               + jnp.dot(agg, wb_ref[...], preferred_element_type=jnp.float32)
               + b1_ref[...])
    out_ref[...] = hb + jnp.dot(y, w2_ref[...],
                                preferred_element_type=jnp.float32) + b2_ref[...]


def _gather_add_call(g_tab, p_tab, col3, row3, colt, rowt, H):
    NW, CH, K = col3.shape
    KT = colt.shape[0] // NW
    EPW = CH * K + KT
    E = NW * EPW
    NB = 3
    CH_MAIN = (CH // NB) * NB
    mesh = plsc.VectorSubcoreMesh(core_axis_name="c", subcore_axis_name="s")
    f32 = jnp.float32

    @functools.partial(
        pl.kernel,
        out_type=jax.ShapeDtypeStruct((E, H), f32),
        mesh=mesh,
        scratch_types=[
            pltpu.VMEM((CH, K), jnp.int32),
            pltpu.VMEM((CH, K), jnp.int32),
            pltpu.VMEM((KT,), jnp.int32),
            pltpu.VMEM((KT,), jnp.int32),
            [pltpu.VMEM((K, H), f32)] * NB,
            [pltpu.VMEM((K, H), f32)] * NB,
            [pltpu.SemaphoreType.DMA] * NB,
            [pltpu.SemaphoreType.DMA] * NB,
            [pltpu.SemaphoreType.DMA] * NB,
        ],
    )
    def gather_k(g_hbm, p_hbm, col_hbm, row_hbm, colt_hbm, rowt_hbm, out_hbm,
                 colv, rowv, colvt, rowvt, gbufs, pbufs, gsems, psems, wsems):
        wid = lax.axis_index("s") * 2 + lax.axis_index("c")
        base_w = wid * EPW
        pltpu.sync_copy(col_hbm.at[wid], colv)
        pltpu.sync_copy(row_hbm.at[wid], rowv)
        pltpu.sync_copy(colt_hbm.at[pl.ds(wid * KT, KT)], colvt)
        pltpu.sync_copy(rowt_hbm.at[pl.ds(wid * KT, KT)], rowvt)

        def issue(c, b):
            pltpu.async_copy(g_hbm.at[colv.at[c]], gbufs[b], gsems[b])
            pltpu.async_copy(p_hbm.at[rowv.at[c]], pbufs[b], psems[b])

        def slot(c, b):
            bz = (b + NB - 1) % NB
            pltpu.make_async_copy(g_hbm.at[pl.ds(0, K)], gbufs[b], gsems[b]).wait()
            pltpu.make_async_copy(p_hbm.at[pl.ds(0, K)], pbufs[b], psems[b]).wait()

            @pl.when(c >= 1)
            def _():
                pltpu.make_async_copy(
                    gbufs[bz], out_hbm.at[pl.ds(0, K)], wsems[bz]).wait()

            @pl.when(c + NB - 1 < CH)
            def _():
                issue(c + NB - 1, bz)

            def edge(e, cc):
                for j in range(H // 16):
                    sl = pl.ds(j * 16, 16)
                    plsc.addupdate(gbufs[b].at[e, sl], pbufs[b][e, sl])
                return cc

            lax.fori_loop(0, K, edge, 0, unroll=4)
            pltpu.async_copy(gbufs[b], out_hbm.at[pl.ds(base_w + c * K, K)],
                             wsems[b])

        for p in range(NB - 1):
            issue(p, p)

        def main(i, cc):
            c = i * NB
            for p in range(NB):
                slot(c + p, p)
            return cc

        lax.fori_loop(0, CH_MAIN // NB, main, 0)
        for c in range(CH_MAIN, CH):
            slot(c, c % NB)
        b_last = (CH - 1) % NB
        pltpu.make_async_copy(gbufs[b_last], out_hbm.at[pl.ds(0, K)],
                              wsems[b_last]).wait()
        bt = (CH) % NB
        pltpu.async_copy(g_hbm.at[colvt], gbufs[bt].at[pl.ds(0, KT)],
                         gsems[bt]).wait()
        pltpu.async_copy(p_hbm.at[rowvt], pbufs[bt].at[pl.ds(0, KT)],
                         psems[bt]).wait()

        def tedge(e, cc):
            for j in range(H // 16):
                sl = pl.ds(j * 16, 16)
                plsc.addupdate(gbufs[bt].at[e, sl], pbufs[bt][e, sl])
            return cc

        lax.fori_loop(0, KT, tedge, 0, unroll=4)
        pltpu.sync_copy(gbufs[bt].at[pl.ds(0, KT)],
                        out_hbm.at[pl.ds(base_w + CH * K, KT)])

    return gather_k(g_tab, p_tab, col3, row3, colt, rowt)


def _scatter_call(feat, col3, N, H):
    NW, CH, K = col3.shape
    EPW = CH * K
    CH_MAIN = (CH // 3) * 3
    FB = 80
    NCH = N // FB
    NFB = -(-NCH // 16)
    mesh = plsc.VectorSubcoreMesh(core_axis_name="c", subcore_axis_name="s")
    f32 = jnp.float32

    @functools.partial(
        pl.kernel,
        out_type=jax.ShapeDtypeStruct((2, N, H), f32),
        mesh=mesh,
        scratch_types=[
            pltpu.VMEM((CH, K), jnp.int32),
            [pltpu.VMEM((K, H), f32)] * 3,
            pltpu.VMEM_SHARED((N, H), f32),
            [pltpu.SemaphoreType.DMA] * 3,
            [pltpu.SemaphoreType.DMA] * 3,
        ],
    )
    def scatter_k(feat_hbm, col_hbm, out_hbm, colv, fbufs, acc, lsems,
                  ssems):
        zbuf = fbufs[0]
        cid = lax.axis_index("c")
        sid = lax.axis_index("s")
        wid = sid * 2 + cid
        base_w = wid * EPW
        z16 = jnp.zeros((16,), f32)

        def zrow(i, c):
            for j in range(H // 16):
                zbuf[i, pl.ds(j * 16, 16)] = z16
            return c

        lax.fori_loop(0, FB, zrow, 0)

        def zacc(b, c):
            g = sid + b * 16

            @pl.when(g < NCH)
            def _():
                pltpu.sync_copy(zbuf, acc.at[pl.ds(g * FB, FB)])

            return c

        lax.fori_loop(0, NFB, zacc, 0)
        pltpu.sync_copy(col_hbm.at[wid], colv)
        plsc.subcore_barrier()

        def load(c, b):
            pltpu.async_copy(feat_hbm.at[pl.ds(base_w + c * K, K)], fbufs[b],
                             lsems[b])

        def slot(c, b):
            bz = (b + 2) % 3
            pltpu.make_async_copy(feat_hbm.at[pl.ds(0, K)], fbufs[b],
                                  lsems[b]).wait()
            pltpu.async_copy(fbufs[b], acc.at[colv.at[c]], ssems[b], add=True)

            @pl.when(c >= 1)
            def _():
                pltpu.make_async_copy(fbufs[bz], acc.at[pl.ds(0, K)],
                                      ssems[bz]).wait()

            @pl.when(c + 2 < CH)
            def _():
                load(c + 2, bz)

        load(0, 0)
        load(1, 1)

        def main(i, cc):
            c = i * 3
            slot(c, 0)
            slot(c + 1, 1)
            slot(c + 2, 2)
            return cc

        lax.fori_loop(0, CH_MAIN // 3, main, 0)
        for c in range(CH_MAIN, CH):
            slot(c, c % 3)
        pltpu.make_async_copy(fbufs[(CH - 1) % 3], acc.at[pl.ds(0, K)],
                              ssems[(CH - 1) % 3]).wait()
        plsc.subcore_barrier()

        def flush(b, c):
            g = sid + b * 16

            @pl.when(g < NCH)
            def _():
                r = g * FB
                pltpu.sync_copy(acc.at[pl.ds(r, FB)], zbuf)
                pltpu.sync_copy(zbuf, out_hbm.at[cid, pl.ds(r, FB)])

            return c

        lax.fori_loop(0, NFB, flush, 0)

    return scatter_k(feat, col3)


def kernel(h, edge_index, edge_attr, edge_W1, edge_b1, bn_gamma, bn_beta,
           edge_W2, edge_b2, node_W1, node_b1, node_W2, node_b2):
    N, D = h.shape
    E, DE = edge_attr.shape
    H = edge_W1.shape[0]
    f32 = jnp.float32

    wg = edge_W1[:, :D].T
    wp = edge_W1[:, D:2 * D].T
    wc = edge_W1[:, 2 * D:].T
    b1 = edge_b1.reshape(1, H)
    w2 = edge_W2.T
    b2 = edge_b2.reshape(1, H)
    nwa = node_W1[:, :D].T
    nwb = node_W1[:, D:].T
    nb1 = node_b1.reshape(1, H)
    nw2 = node_W2.T
    nb2 = node_b2.reshape(1, D)

    TBN = 2000
    BE = 4000
    NW, K = 32, 80
    CH = E // NW // K

    g_tab, p_tab = pl.pallas_call(
        _tables_body,
        grid=(N // TBN,),
        in_specs=[
            pl.BlockSpec((TBN, D), lambda i: (i, 0)),
            pl.BlockSpec((D, H), lambda i: (0, 0)),
            pl.BlockSpec((D, H), lambda i: (0, 0)),
        ],
        out_specs=[
            pl.BlockSpec((TBN, H), lambda i: (i, 0)),
            pl.BlockSpec((TBN, H), lambda i: (i, 0)),
        ],
        out_shape=[jax.ShapeDtypeStruct((N, H), f32),
                   jax.ShapeDtypeStruct((N, H), f32)],
    )(h, wg, wp)

    row3 = edge_index[0].reshape(NW, CH, K)
    col3 = edge_index[1].reshape(NW, CH, K)

    KG = 128
    EPW = E // NW
    CHG = EPW // KG
    KT = EPW - CHG * KG
    row_w = edge_index[0].reshape(NW, EPW)
    col_w = edge_index[1].reshape(NW, EPW)
    row3g = row_w[:, :CHG * KG].reshape(NW, CHG, KG)
    col3g = col_w[:, :CHG * KG].reshape(NW, CHG, KG)
    rowtg = row_w[:, CHG * KG:].reshape(NW * KT)
    coltg = col_w[:, CHG * KG:].reshape(NW * KT)

    gp = _gather_add_call(g_tab, p_tab, col3g, row3g, coltg, rowtg, H)

    stats = pl.pallas_call(
        _stats_body,
        grid=(E // BE,),
        in_specs=[
            pl.BlockSpec((BE, H), lambda i: (i, 0)),
            pl.BlockSpec((BE, DE), lambda i: (i, 0)),
            pl.BlockSpec((DE, H), lambda i: (0, 0)),
            pl.BlockSpec((1, H), lambda i: (0, 0)),
        ],
        out_specs=pl.BlockSpec((2, H), lambda i: (0, 0)),
        out_shape=jax.ShapeDtypeStruct((2, H), f32),
    )(gp, edge_attr, wc, b1)

    ef = pl.pallas_call(
        functools.partial(_edge_body, n_edges=E),
        grid=(E // BE,),
        in_specs=[
            pl.BlockSpec((BE, H), lambda i: (i, 0)),
            pl.BlockSpec((BE, DE), lambda i: (i, 0)),
            pl.BlockSpec((DE, H), lambda i: (0, 0)),
            pl.BlockSpec((1, H), lambda i: (0, 0)),
            pl.BlockSpec((2, H), lambda i: (0, 0)),
            pl.BlockSpec((1, H), lambda i: (0, 0)),
            pl.BlockSpec((1, H), lambda i: (0, 0)),
            pl.BlockSpec((H, H), lambda i: (0, 0)),
            pl.BlockSpec((1, H), lambda i: (0, 0)),
        ],
        out_specs=pl.BlockSpec((BE, H), lambda i: (i, 0)),
        out_shape=jax.ShapeDtypeStruct((E, H), f32),
    )(gp, edge_attr, wc, b1, stats, bn_gamma.reshape(1, H),
      bn_beta.reshape(1, H), w2, b2)

    partials = _scatter_call(ef, col3, N, H)

    out = pl.pallas_call(
        _node_body,
        grid=(N // TBN,),
        in_specs=[
            pl.BlockSpec((TBN, D), lambda i: (i, 0)),
            pl.BlockSpec((TBN, H), lambda i: (i, 0)),
            pl.BlockSpec((TBN, H), lambda i: (i, 0)),
            pl.BlockSpec((D, H), lambda i: (0, 0)),
            pl.BlockSpec((H, H), lambda i: (0, 0)),
            pl.BlockSpec((1, H), lambda i: (0, 0)),
            pl.BlockSpec((H, D), lambda i: (0, 0)),
            pl.BlockSpec((1, D), lambda i: (0, 0)),
        ],
        out_specs=pl.BlockSpec((TBN, D), lambda i: (i, 0)),
        out_shape=jax.ShapeDtypeStruct((N, D), f32),
    )(h, partials[0], partials[1], nwa, nwb, nb1, nw2, nb2)

    return out


# speedup vs baseline: 1.0608x; 1.0608x over previous
import functools

import jax
import jax.numpy as jnp
from jax import lax
from jax.experimental import pallas as pl
from jax.experimental.pallas import tpu as pltpu
from jax.experimental.pallas import tpu_sc as plsc

_INV06 = 1.0 / 0.6


def _ssilu(x):
    return x * jax.nn.sigmoid(x) * _INV06


def _tables_body(h_ref, wg_ref, wp_ref, g_ref, p_ref):
    hb = h_ref[...]
    g_ref[...] = jnp.dot(hb, wg_ref[...], preferred_element_type=jnp.float32)
    p_ref[...] = jnp.dot(hb, wp_ref[...], preferred_element_type=jnp.float32)


def _stats_body(gp_ref, ea_ref, wc_ref, b1_ref, out_ref):
    pre = gp_ref[...] + jnp.dot(ea_ref[...], wc_ref[...],
                                preferred_element_type=jnp.float32) + b1_ref[...]
    x = _ssilu(pre)
    s1 = jnp.sum(x, axis=0, keepdims=True)
    s2 = jnp.sum(x * x, axis=0, keepdims=True)

    @pl.when(pl.program_id(0) == 0)
    def _():
        out_ref[...] = jnp.zeros_like(out_ref)

    out_ref[...] += jnp.concatenate([s1, s2], axis=0)


def _edge_body(gp_ref, ea_ref, wc_ref, b1_ref, st_ref, gam_ref, bet_ref,
               w2_ref, b2_ref, out_ref, *, n_edges):
    mean = st_ref[0:1, :] * (1.0 / n_edges)
    var = st_ref[1:2, :] * (1.0 / n_edges) - mean * mean
    s = gam_ref[...] * jax.lax.rsqrt(var + 1e-5)
    t = bet_ref[...] - mean * s
    pre = gp_ref[...] + jnp.dot(ea_ref[...], wc_ref[...],
                                preferred_element_type=jnp.float32) + b1_ref[...]
    x = _ssilu(pre)
    xn = x * s + t
    out_ref[...] = _ssilu(jnp.dot(xn, w2_ref[...],
                                  preferred_element_type=jnp.float32) + b2_ref[...])


def _node_body(h_ref, p0_ref, p1_ref, wa_ref, wb_ref, b1_ref,
               w2_ref, b2_ref, out_ref):
    hb = h_ref[...]
    agg = (p0_ref[...] + p1_ref[...]) * 0.01
    y = _ssilu(jnp.dot(hb, wa_ref[...], preferred_element_type=jnp.float32)
               + jnp.dot(agg, wb_ref[...], preferred_element_type=jnp.float32)
               + b1_ref[...])
    out_ref[...] = hb + jnp.dot(y, w2_ref[...],
                                preferred_element_type=jnp.float32) + b2_ref[...]


def _gather_add_call(g_tab, p_tab, col3, row3, H):
    NW, CH, K = col3.shape
    EPW = CH * K
    E = NW * EPW
    NB = 4
    CH_MAIN = (CH // NB) * NB
    mesh = plsc.VectorSubcoreMesh(core_axis_name="c", subcore_axis_name="s")
    f32 = jnp.float32

    @functools.partial(
        pl.kernel,
        out_type=jax.ShapeDtypeStruct((E, H), f32),
        mesh=mesh,
        scratch_types=[
            pltpu.VMEM((CH, K), jnp.int32),
            pltpu.VMEM((CH, K), jnp.int32),
            [pltpu.VMEM((K, H), f32)] * NB,
            [pltpu.VMEM((K, H), f32)] * NB,
            [pltpu.SemaphoreType.DMA] * NB,
            [pltpu.SemaphoreType.DMA] * NB,
            [pltpu.SemaphoreType.DMA] * NB,
        ],
    )
    def gather_k(g_hbm, p_hbm, col_hbm, row_hbm, out_hbm, colv, rowv, gbufs,
                 pbufs, gsems, psems, wsems):
        wid = lax.axis_index("s") * 2 + lax.axis_index("c")
        base_w = wid * EPW
        pltpu.sync_copy(col_hbm.at[wid], colv)
        pltpu.sync_copy(row_hbm.at[wid], rowv)

        def issue(c, b):
            pltpu.async_copy(g_hbm.at[colv.at[c]], gbufs[b], gsems[b])
            pltpu.async_copy(p_hbm.at[rowv.at[c]], pbufs[b], psems[b])

        def slot(c, b):
            bz = (b + NB - 1) % NB
            pltpu.make_async_copy(g_hbm.at[pl.ds(0, K)], gbufs[b], gsems[b]).wait()
            pltpu.make_async_copy(p_hbm.at[pl.ds(0, K)], pbufs[b], psems[b]).wait()

            @pl.when(c >= 1)
            def _():
                pltpu.make_async_copy(
                    gbufs[bz], out_hbm.at[pl.ds(0, K)], wsems[bz]).wait()

            @pl.when(c + NB - 1 < CH)
            def _():
                issue(c + NB - 1, bz)

            def edge(e, cc):
                for j in range(H // 16):
                    sl = pl.ds(j * 16, 16)
                    plsc.addupdate(gbufs[b].at[e, sl], pbufs[b][e, sl])
                return cc

            lax.fori_loop(0, K, edge, 0, unroll=4)
            pltpu.async_copy(gbufs[b], out_hbm.at[pl.ds(base_w + c * K, K)],
                             wsems[b])

        for p in range(NB - 1):
            issue(p, p)

        def main(i, cc):
            c = i * NB
            for p in range(NB):
                slot(c + p, p)
            return cc

        lax.fori_loop(0, CH_MAIN // NB, main, 0)
        for c in range(CH_MAIN, CH):
            slot(c, c % NB)
        pltpu.make_async_copy(gbufs[(CH - 1) % NB], out_hbm.at[pl.ds(0, K)],
                              wsems[(CH - 1) % NB]).wait()

    return gather_k(g_tab, p_tab, col3, row3)


def _scatter_call(feat, col3, N, H):
    NW, CH, K = col3.shape
    EPW = CH * K
    CH_MAIN = (CH // 3) * 3
    FB = 80
    NCH = N // FB
    NFB = -(-NCH // 16)
    mesh = plsc.VectorSubcoreMesh(core_axis_name="c", subcore_axis_name="s")
    f32 = jnp.float32

    @functools.partial(
        pl.kernel,
        out_type=jax.ShapeDtypeStruct((2, N, H), f32),
        mesh=mesh,
        scratch_types=[
            pltpu.VMEM((CH, K), jnp.int32),
            [pltpu.VMEM((K, H), f32)] * 3,
            pltpu.VMEM_SHARED((N, H), f32),
            [pltpu.SemaphoreType.DMA] * 3,
            [pltpu.SemaphoreType.DMA] * 3,
        ],
    )
    def scatter_k(feat_hbm, col_hbm, out_hbm, colv, fbufs, acc, lsems,
                  ssems):
        zbuf = fbufs[0]
        cid = lax.axis_index("c")
        sid = lax.axis_index("s")
        wid = sid * 2 + cid
        base_w = wid * EPW
        z16 = jnp.zeros((16,), f32)

        def zrow(i, c):
            for j in range(H // 16):
                zbuf[i, pl.ds(j * 16, 16)] = z16
            return c

        lax.fori_loop(0, FB, zrow, 0)

        def zacc(b, c):
            g = sid + b * 16

            @pl.when(g < NCH)
            def _():
                pltpu.sync_copy(zbuf, acc.at[pl.ds(g * FB, FB)])

            return c

        lax.fori_loop(0, NFB, zacc, 0)
        pltpu.sync_copy(col_hbm.at[wid], colv)
        plsc.subcore_barrier()

        def load(c, b):
            pltpu.async_copy(feat_hbm.at[pl.ds(base_w + c * K, K)], fbufs[b],
                             lsems[b])

        def slot(c, b):
            bz = (b + 2) % 3
            pltpu.make_async_copy(feat_hbm.at[pl.ds(0, K)], fbufs[b],
                                  lsems[b]).wait()
            pltpu.async_copy(fbufs[b], acc.at[colv.at[c]], ssems[b], add=True)

            @pl.when(c >= 1)
            def _():
                pltpu.make_async_copy(fbufs[bz], acc.at[pl.ds(0, K)],
                                      ssems[bz]).wait()

            @pl.when(c + 2 < CH)
            def _():
                load(c + 2, bz)

        load(0, 0)
        load(1, 1)

        def main(i, cc):
            c = i * 3
            slot(c, 0)
            slot(c + 1, 1)
            slot(c + 2, 2)
            return cc

        lax.fori_loop(0, CH_MAIN // 3, main, 0)
        for c in range(CH_MAIN, CH):
            slot(c, c % 3)
        pltpu.make_async_copy(fbufs[(CH - 1) % 3], acc.at[pl.ds(0, K)],
                              ssems[(CH - 1) % 3]).wait()
        plsc.subcore_barrier()

        def flush(b, c):
            g = sid + b * 16

            @pl.when(g < NCH)
            def _():
                r = g * FB
                pltpu.sync_copy(acc.at[pl.ds(r, FB)], zbuf)
                pltpu.sync_copy(zbuf, out_hbm.at[cid, pl.ds(r, FB)])

            return c

        lax.fori_loop(0, NFB, flush, 0)

    return scatter_k(feat, col3)


def kernel(h, edge_index, edge_attr, edge_W1, edge_b1, bn_gamma, bn_beta,
           edge_W2, edge_b2, node_W1, node_b1, node_W2, node_b2):
    N, D = h.shape
    E, DE = edge_attr.shape
    H = edge_W1.shape[0]
    f32 = jnp.float32

    wg = edge_W1[:, :D].T
    wp = edge_W1[:, D:2 * D].T
    wc = edge_W1[:, 2 * D:].T
    b1 = edge_b1.reshape(1, H)
    w2 = edge_W2.T
    b2 = edge_b2.reshape(1, H)
    nwa = node_W1[:, :D].T
    nwb = node_W1[:, D:].T
    nb1 = node_b1.reshape(1, H)
    nw2 = node_W2.T
    nb2 = node_b2.reshape(1, D)

    TBN = 2000
    BE = 8000
    NW, K = 32, 80
    CH = E // NW // K

    g_tab, p_tab = pl.pallas_call(
        _tables_body,
        grid=(N // TBN,),
        in_specs=[
            pl.BlockSpec((TBN, D), lambda i: (i, 0)),
            pl.BlockSpec((D, H), lambda i: (0, 0)),
            pl.BlockSpec((D, H), lambda i: (0, 0)),
        ],
        out_specs=[
            pl.BlockSpec((TBN, H), lambda i: (i, 0)),
            pl.BlockSpec((TBN, H), lambda i: (i, 0)),
        ],
        out_shape=[jax.ShapeDtypeStruct((N, H), f32),
                   jax.ShapeDtypeStruct((N, H), f32)],
    )(h, wg, wp)

    row3 = edge_index[0].reshape(NW, CH, K)
    col3 = edge_index[1].reshape(NW, CH, K)

    gp = _gather_add_call(g_tab, p_tab, col3, row3, H)

    stats = pl.pallas_call(
        _stats_body,
        grid=(E // BE,),
        in_specs=[
            pl.BlockSpec((BE, H), lambda i: (i, 0)),
            pl.BlockSpec((BE, DE), lambda i: (i, 0)),
            pl.BlockSpec((DE, H), lambda i: (0, 0)),
            pl.BlockSpec((1, H), lambda i: (0, 0)),
        ],
        out_specs=pl.BlockSpec((2, H), lambda i: (0, 0)),
        out_shape=jax.ShapeDtypeStruct((2, H), f32),
    )(gp, edge_attr, wc, b1)

    ef = pl.pallas_call(
        functools.partial(_edge_body, n_edges=E),
        grid=(E // BE,),
        in_specs=[
            pl.BlockSpec((BE, H), lambda i: (i, 0)),
            pl.BlockSpec((BE, DE), lambda i: (i, 0)),
            pl.BlockSpec((DE, H), lambda i: (0, 0)),
            pl.BlockSpec((1, H), lambda i: (0, 0)),
            pl.BlockSpec((2, H), lambda i: (0, 0)),
            pl.BlockSpec((1, H), lambda i: (0, 0)),
            pl.BlockSpec((1, H), lambda i: (0, 0)),
            pl.BlockSpec((H, H), lambda i: (0, 0)),
            pl.BlockSpec((1, H), lambda i: (0, 0)),
        ],
        out_specs=pl.BlockSpec((BE, H), lambda i: (i, 0)),
        out_shape=jax.ShapeDtypeStruct((E, H), f32),
    )(gp, edge_attr, wc, b1, stats, bn_gamma.reshape(1, H),
      bn_beta.reshape(1, H), w2, b2)

    partials = _scatter_call(ef, col3, N, H)

    out = pl.pallas_call(
        _node_body,
        grid=(N // TBN,),
        in_specs=[
            pl.BlockSpec((TBN, D), lambda i: (i, 0)),
            pl.BlockSpec((TBN, H), lambda i: (i, 0)),
            pl.BlockSpec((TBN, H), lambda i: (i, 0)),
            pl.BlockSpec((D, H), lambda i: (0, 0)),
            pl.BlockSpec((H, H), lambda i: (0, 0)),
            pl.BlockSpec((1, H), lambda i: (0, 0)),
            pl.BlockSpec((H, D), lambda i: (0, 0)),
            pl.BlockSpec((1, D), lambda i: (0, 0)),
        ],
        out_specs=pl.BlockSpec((TBN, D), lambda i: (i, 0)),
        out_shape=jax.ShapeDtypeStruct((N, D), f32),
    )(h, partials[0], partials[1], nwa, nwb, nb1, nw2, nb2)

    return out


# speedup vs baseline: 1.0730x; 1.0116x over previous
import functools

import jax
import jax.numpy as jnp
from jax import lax
from jax.experimental import pallas as pl
from jax.experimental.pallas import tpu as pltpu
from jax.experimental.pallas import tpu_sc as plsc

_INV06 = 1.0 / 0.6


def _ssilu(x):
    return x * jax.nn.sigmoid(x) * _INV06


def _tables_body(h_ref, wg_ref, wp_ref, g_ref, p_ref):
    hb = h_ref[...]
    g_ref[...] = jnp.dot(hb, wg_ref[...], preferred_element_type=jnp.float32)
    p_ref[...] = jnp.dot(hb, wp_ref[...], preferred_element_type=jnp.float32)


def _stats_body(gp_ref, ea_ref, wc_ref, b1_ref, out_ref):
    pre = gp_ref[...] + jnp.dot(ea_ref[...], wc_ref[...],
                                preferred_element_type=jnp.float32) + b1_ref[...]
    x = _ssilu(pre)
    s1 = jnp.sum(x, axis=0, keepdims=True)
    s2 = jnp.sum(x * x, axis=0, keepdims=True)

    @pl.when(pl.program_id(0) == 0)
    def _():
        out_ref[...] = jnp.zeros_like(out_ref)

    out_ref[...] += jnp.concatenate([s1, s2], axis=0)


def _edge_body(gp_ref, ea_ref, wc_ref, b1_ref, st_ref, gam_ref, bet_ref,
               w2_ref, b2_ref, out_ref, *, n_edges):
    mean = st_ref[0:1, :] * (1.0 / n_edges)
    var = st_ref[1:2, :] * (1.0 / n_edges) - mean * mean
    s = gam_ref[...] * jax.lax.rsqrt(var + 1e-5)
    t = bet_ref[...] - mean * s
    pre = gp_ref[...] + jnp.dot(ea_ref[...], wc_ref[...],
                                preferred_element_type=jnp.float32) + b1_ref[...]
    x = _ssilu(pre)
    xn = x * s + t
    out_ref[...] = _ssilu(jnp.dot(xn, w2_ref[...],
                                  preferred_element_type=jnp.float32) + b2_ref[...])


def _node_body(h_ref, p0_ref, p1_ref, wa_ref, wb_ref, b1_ref,
               w2_ref, b2_ref, out_ref):
    hb = h_ref[...]
    agg = (p0_ref[...] + p1_ref[...]) * 0.01
    y = _ssilu(jnp.dot(hb, wa_ref[...], preferred_element_type=jnp.float32)
               + jnp.dot(agg, wb_ref[...], preferred_element_type=jnp.float32)
               + b1_ref[...])
    out_ref[...] = hb + jnp.dot(y, w2_ref[...],
                                preferred_element_type=jnp.float32) + b2_ref[...]


def _gather_add_call(g_tab, p_tab, col3, row3, H):
    NW, CH, K = col3.shape
    EPW = CH * K
    E = NW * EPW
    NB = 4
    CH_MAIN = (CH // NB) * NB
    mesh = plsc.VectorSubcoreMesh(core_axis_name="c", subcore_axis_name="s")
    f32 = jnp.float32

    @functools.partial(
        pl.kernel,
        out_type=jax.ShapeDtypeStruct((E, H), f32),
        mesh=mesh,
        scratch_types=[
            pltpu.VMEM((CH, K), jnp.int32),
            pltpu.VMEM((CH, K), jnp.int32),
            [pltpu.VMEM((K, H), f32)] * NB,
            [pltpu.VMEM((K, H), f32)] * NB,
            [pltpu.SemaphoreType.DMA] * NB,
            [pltpu.SemaphoreType.DMA] * NB,
            [pltpu.SemaphoreType.DMA] * NB,
        ],
    )
    def gather_k(g_hbm, p_hbm, col_hbm, row_hbm, out_hbm, colv, rowv, gbufs,
                 pbufs, gsems, psems, wsems):
        wid = lax.axis_index("s") * 2 + lax.axis_index("c")
        base_w = wid * EPW
        pltpu.sync_copy(col_hbm.at[wid], colv)
        pltpu.sync_copy(row_hbm.at[wid], rowv)

        def issue(c, b):
            pltpu.async_copy(g_hbm.at[colv.at[c]], gbufs[b], gsems[b])
            pltpu.async_copy(p_hbm.at[rowv.at[c]], pbufs[b], psems[b])

        def slot(c, b):
            bz = (b + NB - 1) % NB
            pltpu.make_async_copy(g_hbm.at[pl.ds(0, K)], gbufs[b], gsems[b]).wait()
            pltpu.make_async_copy(p_hbm.at[pl.ds(0, K)], pbufs[b], psems[b]).wait()

            @pl.when(c >= 1)
            def _():
                pltpu.make_async_copy(
                    gbufs[bz], out_hbm.at[pl.ds(0, K)], wsems[bz]).wait()

            @pl.when(c + NB - 1 < CH)
            def _():
                issue(c + NB - 1, bz)

            def edge(e, cc):
                for j in range(H // 16):
                    sl = pl.ds(j * 16, 16)
                    plsc.addupdate(gbufs[b].at[e, sl], pbufs[b][e, sl])
                return cc

            lax.fori_loop(0, K, edge, 0, unroll=4)
            pltpu.async_copy(gbufs[b], out_hbm.at[pl.ds(base_w + c * K, K)],
                             wsems[b])

        for p in range(NB - 1):
            issue(p, p)

        def main(i, cc):
            c = i * NB
            for p in range(NB):
                slot(c + p, p)
            return cc

        lax.fori_loop(0, CH_MAIN // NB, main, 0)
        for c in range(CH_MAIN, CH):
            slot(c, c % NB)
        pltpu.make_async_copy(gbufs[(CH - 1) % NB], out_hbm.at[pl.ds(0, K)],
                              wsems[(CH - 1) % NB]).wait()

    return gather_k(g_tab, p_tab, col3, row3)


def _scatter_call(feat, col3, N, H):
    NW, CH, K = col3.shape
    EPW = CH * K
    CH_MAIN = (CH // 3) * 3
    FB = 80
    NCH = N // FB
    NFB = -(-NCH // 16)
    mesh = plsc.VectorSubcoreMesh(core_axis_name="c", subcore_axis_name="s")
    f32 = jnp.float32

    @functools.partial(
        pl.kernel,
        out_type=jax.ShapeDtypeStruct((2, N, H), f32),
        mesh=mesh,
        scratch_types=[
            pltpu.VMEM((CH, K), jnp.int32),
            [pltpu.VMEM((K, H), f32)] * 3,
            pltpu.VMEM_SHARED((N, H), f32),
            [pltpu.SemaphoreType.DMA] * 3,
            [pltpu.SemaphoreType.DMA] * 3,
        ],
    )
    def scatter_k(feat_hbm, col_hbm, out_hbm, colv, fbufs, acc, lsems,
                  ssems):
        zbuf = fbufs[0]
        cid = lax.axis_index("c")
        sid = lax.axis_index("s")
        wid = sid * 2 + cid
        base_w = wid * EPW
        z16 = jnp.zeros((16,), f32)

        def zrow(i, c):
            for j in range(H // 16):
                zbuf[i, pl.ds(j * 16, 16)] = z16
            return c

        lax.fori_loop(0, FB, zrow, 0)

        def zacc(b, c):
            g = sid + b * 16

            @pl.when(g < NCH)
            def _():
                pltpu.sync_copy(zbuf, acc.at[pl.ds(g * FB, FB)])

            return c

        lax.fori_loop(0, NFB, zacc, 0)
        pltpu.sync_copy(col_hbm.at[wid], colv)
        plsc.subcore_barrier()

        def load(c, b):
            pltpu.async_copy(feat_hbm.at[pl.ds(base_w + c * K, K)], fbufs[b],
                             lsems[b])

        def slot(c, b):
            bz = (b + 2) % 3
            pltpu.make_async_copy(feat_hbm.at[pl.ds(0, K)], fbufs[b],
                                  lsems[b]).wait()
            pltpu.async_copy(fbufs[b], acc.at[colv.at[c]], ssems[b], add=True)

            @pl.when(c >= 1)
            def _():
                pltpu.make_async_copy(fbufs[bz], acc.at[pl.ds(0, K)],
                                      ssems[bz]).wait()

            @pl.when(c + 2 < CH)
            def _():
                load(c + 2, bz)

        load(0, 0)
        load(1, 1)

        def main(i, cc):
            c = i * 3
            slot(c, 0)
            slot(c + 1, 1)
            slot(c + 2, 2)
            return cc

        lax.fori_loop(0, CH_MAIN // 3, main, 0)
        for c in range(CH_MAIN, CH):
            slot(c, c % 3)
        pltpu.make_async_copy(fbufs[(CH - 1) % 3], acc.at[pl.ds(0, K)],
                              ssems[(CH - 1) % 3]).wait()
        plsc.subcore_barrier()

        def flush(b, c):
            g = sid + b * 16

            @pl.when(g < NCH)
            def _():
                r = g * FB
                pltpu.sync_copy(acc.at[pl.ds(r, FB)], zbuf)
                pltpu.sync_copy(zbuf, out_hbm.at[cid, pl.ds(r, FB)])

            return c

        lax.fori_loop(0, NFB, flush, 0)

    return scatter_k(feat, col3)


def kernel(h, edge_index, edge_attr, edge_W1, edge_b1, bn_gamma, bn_beta,
           edge_W2, edge_b2, node_W1, node_b1, node_W2, node_b2):
    N, D = h.shape
    E, DE = edge_attr.shape
    H = edge_W1.shape[0]
    f32 = jnp.float32

    wg = edge_W1[:, :D].T
    wp = edge_W1[:, D:2 * D].T
    wc = edge_W1[:, 2 * D:].T
    b1 = edge_b1.reshape(1, H)
    w2 = edge_W2.T
    b2 = edge_b2.reshape(1, H)
    nwa = node_W1[:, :D].T
    nwb = node_W1[:, D:].T
    nb1 = node_b1.reshape(1, H)
    nw2 = node_W2.T
    nb2 = node_b2.reshape(1, D)

    TBN = 2000
    BE = 16000
    NW, K = 32, 80
    CH = E // NW // K

    g_tab, p_tab = pl.pallas_call(
        _tables_body,
        grid=(N // TBN,),
        in_specs=[
            pl.BlockSpec((TBN, D), lambda i: (i, 0)),
            pl.BlockSpec((D, H), lambda i: (0, 0)),
            pl.BlockSpec((D, H), lambda i: (0, 0)),
        ],
        out_specs=[
            pl.BlockSpec((TBN, H), lambda i: (i, 0)),
            pl.BlockSpec((TBN, H), lambda i: (i, 0)),
        ],
        out_shape=[jax.ShapeDtypeStruct((N, H), f32),
                   jax.ShapeDtypeStruct((N, H), f32)],
    )(h, wg, wp)

    row3 = edge_index[0].reshape(NW, CH, K)
    col3 = edge_index[1].reshape(NW, CH, K)

    gp = _gather_add_call(g_tab, p_tab, col3, row3, H)

    stats = pl.pallas_call(
        _stats_body,
        grid=(E // BE,),
        in_specs=[
            pl.BlockSpec((BE, H), lambda i: (i, 0)),
            pl.BlockSpec((BE, DE), lambda i: (i, 0)),
            pl.BlockSpec((DE, H), lambda i: (0, 0)),
            pl.BlockSpec((1, H), lambda i: (0, 0)),
        ],
        out_specs=pl.BlockSpec((2, H), lambda i: (0, 0)),
        out_shape=jax.ShapeDtypeStruct((2, H), f32),
    )(gp, edge_attr, wc, b1)

    ef = pl.pallas_call(
        functools.partial(_edge_body, n_edges=E),
        grid=(E // BE,),
        in_specs=[
            pl.BlockSpec((BE, H), lambda i: (i, 0)),
            pl.BlockSpec((BE, DE), lambda i: (i, 0)),
            pl.BlockSpec((DE, H), lambda i: (0, 0)),
            pl.BlockSpec((1, H), lambda i: (0, 0)),
            pl.BlockSpec((2, H), lambda i: (0, 0)),
            pl.BlockSpec((1, H), lambda i: (0, 0)),
            pl.BlockSpec((1, H), lambda i: (0, 0)),
            pl.BlockSpec((H, H), lambda i: (0, 0)),
            pl.BlockSpec((1, H), lambda i: (0, 0)),
        ],
        out_specs=pl.BlockSpec((BE, H), lambda i: (i, 0)),
        out_shape=jax.ShapeDtypeStruct((E, H), f32),
    )(gp, edge_attr, wc, b1, stats, bn_gamma.reshape(1, H),
      bn_beta.reshape(1, H), w2, b2)

    partials = _scatter_call(ef, col3, N, H)

    out = pl.pallas_call(
        _node_body,
        grid=(N // TBN,),
        in_specs=[
            pl.BlockSpec((TBN, D), lambda i: (i, 0)),
            pl.BlockSpec((TBN, H), lambda i: (i, 0)),
            pl.BlockSpec((TBN, H), lambda i: (i, 0)),
            pl.BlockSpec((D, H), lambda i: (0, 0)),
            pl.BlockSpec((H, H), lambda i: (0, 0)),
            pl.BlockSpec((1, H), lambda i: (0, 0)),
            pl.BlockSpec((H, D), lambda i: (0, 0)),
            pl.BlockSpec((1, D), lambda i: (0, 0)),
        ],
        out_specs=pl.BlockSpec((TBN, D), lambda i: (i, 0)),
        out_shape=jax.ShapeDtypeStruct((N, D), f32),
    )(h, partials[0], partials[1], nwa, nwb, nb1, nw2, nb2)

    return out


# speedup vs baseline: 1.0776x; 1.0043x over previous
import functools

import jax
import jax.numpy as jnp
from jax import lax
from jax.experimental import pallas as pl
from jax.experimental.pallas import tpu as pltpu
from jax.experimental.pallas import tpu_sc as plsc

_INV06 = 1.0 / 0.6


def _ssilu(x):
    return x * jax.nn.sigmoid(x) * _INV06


def _tables_body(h_ref, wg_ref, wp_ref, g_ref, p_ref):
    hb = h_ref[...]
    g_ref[...] = jnp.dot(hb, wg_ref[...], preferred_element_type=jnp.float32)
    p_ref[...] = jnp.dot(hb, wp_ref[...], preferred_element_type=jnp.float32)


def _stats_body(gp_ref, ea_ref, wc_ref, b1_ref, out_ref):
    pre = gp_ref[...] + jnp.dot(ea_ref[...], wc_ref[...],
                                preferred_element_type=jnp.float32) + b1_ref[...]
    x = _ssilu(pre)
    s1 = jnp.sum(x, axis=0, keepdims=True)
    s2 = jnp.sum(x * x, axis=0, keepdims=True)

    @pl.when(pl.program_id(0) == 0)
    def _():
        out_ref[...] = jnp.zeros_like(out_ref)

    out_ref[...] += jnp.concatenate([s1, s2], axis=0)


def _edge_body(gp_ref, ea_ref, wc_ref, b1_ref, st_ref, gam_ref, bet_ref,
               w2_ref, b2_ref, out_ref, *, n_edges):
    mean = st_ref[0:1, :] * (1.0 / n_edges)
    var = st_ref[1:2, :] * (1.0 / n_edges) - mean * mean
    s = gam_ref[...] * jax.lax.rsqrt(var + 1e-5)
    t = bet_ref[...] - mean * s
    pre = gp_ref[...] + jnp.dot(ea_ref[...], wc_ref[...],
                                preferred_element_type=jnp.float32) + b1_ref[...]
    x = _ssilu(pre)
    xn = x * s + t
    out_ref[...] = _ssilu(jnp.dot(xn, w2_ref[...],
                                  preferred_element_type=jnp.float32) + b2_ref[...])


def _node_body(h_ref, p0_ref, p1_ref, wa_ref, wb_ref, b1_ref,
               w2_ref, b2_ref, out_ref):
    hb = h_ref[...]
    agg = (p0_ref[...] + p1_ref[...]) * 0.01
    y = _ssilu(jnp.dot(hb, wa_ref[...], preferred_element_type=jnp.float32)
               + jnp.dot(agg, wb_ref[...], preferred_element_type=jnp.float32)
               + b1_ref[...])
    out_ref[...] = hb + jnp.dot(y, w2_ref[...],
                                preferred_element_type=jnp.float32) + b2_ref[...]


def _gather_add_call(g_tab, p_tab, col3, row3, H):
    NW, CH, K = col3.shape
    EPW = CH * K
    E = NW * EPW
    NB = 4
    CH_MAIN = (CH // NB) * NB
    mesh = plsc.VectorSubcoreMesh(core_axis_name="c", subcore_axis_name="s")
    f32 = jnp.float32

    @functools.partial(
        pl.kernel,
        out_type=jax.ShapeDtypeStruct((E, H), f32),
        mesh=mesh,
        scratch_types=[
            pltpu.VMEM((CH, K), jnp.int32),
            pltpu.VMEM((CH, K), jnp.int32),
            [pltpu.VMEM((K, H), f32)] * NB,
            [pltpu.VMEM((K, H), f32)] * NB,
            [pltpu.SemaphoreType.DMA] * NB,
            [pltpu.SemaphoreType.DMA] * NB,
            [pltpu.SemaphoreType.DMA] * NB,
        ],
    )
    def gather_k(g_hbm, p_hbm, col_hbm, row_hbm, out_hbm, colv, rowv, gbufs,
                 pbufs, gsems, psems, wsems):
        wid = lax.axis_index("s") * 2 + lax.axis_index("c")
        base_w = wid * EPW
        pltpu.sync_copy(col_hbm.at[wid], colv)
        pltpu.sync_copy(row_hbm.at[wid], rowv)

        def issue(c, b):
            pltpu.async_copy(g_hbm.at[colv.at[c]], gbufs[b], gsems[b])
            pltpu.async_copy(p_hbm.at[rowv.at[c]], pbufs[b], psems[b])

        def slot(c, b):
            bz = (b + NB - 1) % NB
            pltpu.make_async_copy(g_hbm.at[pl.ds(0, K)], gbufs[b], gsems[b]).wait()
            pltpu.make_async_copy(p_hbm.at[pl.ds(0, K)], pbufs[b], psems[b]).wait()

            @pl.when(c >= 1)
            def _():
                pltpu.make_async_copy(
                    gbufs[bz], out_hbm.at[pl.ds(0, K)], wsems[bz]).wait()

            @pl.when(c + NB - 1 < CH)
            def _():
                issue(c + NB - 1, bz)

            def edge(e, cc):
                for j in range(H // 16):
                    sl = pl.ds(j * 16, 16)
                    plsc.addupdate(gbufs[b].at[e, sl], pbufs[b][e, sl])
                return cc

            lax.fori_loop(0, K, edge, 0, unroll=4)
            pltpu.async_copy(gbufs[b], out_hbm.at[pl.ds(base_w + c * K, K)],
                             wsems[b])

        for p in range(NB - 1):
            issue(p, p)

        def main(i, cc):
            c = i * NB
            for p in range(NB):
                slot(c + p, p)
            return cc

        lax.fori_loop(0, CH_MAIN // NB, main, 0)
        for c in range(CH_MAIN, CH):
            slot(c, c % NB)
        pltpu.make_async_copy(gbufs[(CH - 1) % NB], out_hbm.at[pl.ds(0, K)],
                              wsems[(CH - 1) % NB]).wait()

    return gather_k(g_tab, p_tab, col3, row3)


def _scatter_call(feat, col3, N, H):
    NW, CH, K = col3.shape
    EPW = CH * K
    CH_MAIN = (CH // 3) * 3
    FB = 80
    NCH = N // FB
    NFB = -(-NCH // 16)
    mesh = plsc.VectorSubcoreMesh(core_axis_name="c", subcore_axis_name="s")
    f32 = jnp.float32

    @functools.partial(
        pl.kernel,
        out_type=jax.ShapeDtypeStruct((2, N, H), f32),
        mesh=mesh,
        scratch_types=[
            pltpu.VMEM((CH, K), jnp.int32),
            [pltpu.VMEM((K, H), f32)] * 3,
            pltpu.VMEM_SHARED((N, H), f32),
            [pltpu.SemaphoreType.DMA] * 3,
            [pltpu.SemaphoreType.DMA] * 3,
        ],
    )
    def scatter_k(feat_hbm, col_hbm, out_hbm, colv, fbufs, acc, lsems,
                  ssems):
        zbuf = fbufs[0]
        cid = lax.axis_index("c")
        sid = lax.axis_index("s")
        wid = sid * 2 + cid
        base_w = wid * EPW
        z16 = jnp.zeros((16,), f32)

        def zrow(i, c):
            for j in range(H // 16):
                zbuf[i, pl.ds(j * 16, 16)] = z16
            return c

        lax.fori_loop(0, FB, zrow, 0)

        def zacc(b, c):
            g = sid + b * 16

            @pl.when(g < NCH)
            def _():
                pltpu.sync_copy(zbuf, acc.at[pl.ds(g * FB, FB)])

            return c

        lax.fori_loop(0, NFB, zacc, 0)
        pltpu.sync_copy(col_hbm.at[wid], colv)
        plsc.subcore_barrier()

        def load(c, b):
            pltpu.async_copy(feat_hbm.at[pl.ds(base_w + c * K, K)], fbufs[b],
                             lsems[b])

        def slot(c, b):
            bz = (b + 2) % 3
            pltpu.make_async_copy(feat_hbm.at[pl.ds(0, K)], fbufs[b],
                                  lsems[b]).wait()
            pltpu.async_copy(fbufs[b], acc.at[colv.at[c]], ssems[b], add=True)

            @pl.when(c >= 1)
            def _():
                pltpu.make_async_copy(fbufs[bz], acc.at[pl.ds(0, K)],
                                      ssems[bz]).wait()

            @pl.when(c + 2 < CH)
            def _():
                load(c + 2, bz)

        load(0, 0)
        load(1, 1)

        def main(i, cc):
            c = i * 3
            slot(c, 0)
            slot(c + 1, 1)
            slot(c + 2, 2)
            return cc

        lax.fori_loop(0, CH_MAIN // 3, main, 0)
        for c in range(CH_MAIN, CH):
            slot(c, c % 3)
        pltpu.make_async_copy(fbufs[(CH - 1) % 3], acc.at[pl.ds(0, K)],
                              ssems[(CH - 1) % 3]).wait()
        plsc.subcore_barrier()

        def flush(b, c):
            g = sid + b * 16

            @pl.when(g < NCH)
            def _():
                r = g * FB
                pltpu.sync_copy(acc.at[pl.ds(r, FB)], zbuf)
                pltpu.sync_copy(zbuf, out_hbm.at[cid, pl.ds(r, FB)])

            return c

        lax.fori_loop(0, NFB, flush, 0)

    return scatter_k(feat, col3)


def kernel(h, edge_index, edge_attr, edge_W1, edge_b1, bn_gamma, bn_beta,
           edge_W2, edge_b2, node_W1, node_b1, node_W2, node_b2):
    N, D = h.shape
    E, DE = edge_attr.shape
    H = edge_W1.shape[0]
    f32 = jnp.float32

    wg = edge_W1[:, :D].T
    wp = edge_W1[:, D:2 * D].T
    wc = edge_W1[:, 2 * D:].T
    b1 = edge_b1.reshape(1, H)
    w2 = edge_W2.T
    b2 = edge_b2.reshape(1, H)
    nwa = node_W1[:, :D].T
    nwb = node_W1[:, D:].T
    nb1 = node_b1.reshape(1, H)
    nw2 = node_W2.T
    nb2 = node_b2.reshape(1, D)

    TBN = 5000
    BE = 16000
    NW, K = 32, 80
    CH = E // NW // K

    g_tab, p_tab = pl.pallas_call(
        _tables_body,
        grid=(N // TBN,),
        in_specs=[
            pl.BlockSpec((TBN, D), lambda i: (i, 0)),
            pl.BlockSpec((D, H), lambda i: (0, 0)),
            pl.BlockSpec((D, H), lambda i: (0, 0)),
        ],
        out_specs=[
            pl.BlockSpec((TBN, H), lambda i: (i, 0)),
            pl.BlockSpec((TBN, H), lambda i: (i, 0)),
        ],
        out_shape=[jax.ShapeDtypeStruct((N, H), f32),
                   jax.ShapeDtypeStruct((N, H), f32)],
    )(h, wg, wp)

    row3 = edge_index[0].reshape(NW, CH, K)
    col3 = edge_index[1].reshape(NW, CH, K)

    gp = _gather_add_call(g_tab, p_tab, col3, row3, H)

    stats = pl.pallas_call(
        _stats_body,
        grid=(E // BE,),
        in_specs=[
            pl.BlockSpec((BE, H), lambda i: (i, 0)),
            pl.BlockSpec((BE, DE), lambda i: (i, 0)),
            pl.BlockSpec((DE, H), lambda i: (0, 0)),
            pl.BlockSpec((1, H), lambda i: (0, 0)),
        ],
        out_specs=pl.BlockSpec((2, H), lambda i: (0, 0)),
        out_shape=jax.ShapeDtypeStruct((2, H), f32),
    )(gp, edge_attr, wc, b1)

    ef = pl.pallas_call(
        functools.partial(_edge_body, n_edges=E),
        grid=(E // BE,),
        in_specs=[
            pl.BlockSpec((BE, H), lambda i: (i, 0)),
            pl.BlockSpec((BE, DE), lambda i: (i, 0)),
            pl.BlockSpec((DE, H), lambda i: (0, 0)),
            pl.BlockSpec((1, H), lambda i: (0, 0)),
            pl.BlockSpec((2, H), lambda i: (0, 0)),
            pl.BlockSpec((1, H), lambda i: (0, 0)),
            pl.BlockSpec((1, H), lambda i: (0, 0)),
            pl.BlockSpec((H, H), lambda i: (0, 0)),
            pl.BlockSpec((1, H), lambda i: (0, 0)),
        ],
        out_specs=pl.BlockSpec((BE, H), lambda i: (i, 0)),
        out_shape=jax.ShapeDtypeStruct((E, H), f32),
    )(gp, edge_attr, wc, b1, stats, bn_gamma.reshape(1, H),
      bn_beta.reshape(1, H), w2, b2)

    partials = _scatter_call(ef, col3, N, H)

    out = pl.pallas_call(
        _node_body,
        grid=(N // TBN,),
        in_specs=[
            pl.BlockSpec((TBN, D), lambda i: (i, 0)),
            pl.BlockSpec((TBN, H), lambda i: (i, 0)),
            pl.BlockSpec((TBN, H), lambda i: (i, 0)),
            pl.BlockSpec((D, H), lambda i: (0, 0)),
            pl.BlockSpec((H, H), lambda i: (0, 0)),
            pl.BlockSpec((1, H), lambda i: (0, 0)),
            pl.BlockSpec((H, D), lambda i: (0, 0)),
            pl.BlockSpec((1, D), lambda i: (0, 0)),
        ],
        out_specs=pl.BlockSpec((TBN, D), lambda i: (i, 0)),
        out_shape=jax.ShapeDtypeStruct((N, D), f32),
    )(h, partials[0], partials[1], nwa, nwb, nb1, nw2, nb2)

    return out
